# baseline (speedup 1.0000x reference)
import functools

import jax
import jax.numpy as jnp
from jax import lax
from jax.experimental import pallas as pl
from jax.experimental.pallas import tpu as pltpu
from jax.experimental.pallas import tpu_sc as plsc

N = 50000
E = 800000
G = 1024
NF = 64
EF = 16
GF = 256
FP = 4096

NC = 2
NS = 16
NW = NC * NS
L = 16

NPAD = 50176
EPAD = 819200
EW = EPAD // NW
EB = 2560
NEG = EB // L
GB = 640
SROW = NPAD // NS
GA = 1152
GROW = GA // NS
NB = NPAD // NW
NBB = 224
DUMMY_DST = 50100
DUMMY_G = 1024
CLAMP = 45.0
NCHUNK = GF // L

_mesh = plsc.VectorSubcoreMesh(core_axis_name="c", subcore_axis_name="s",
                               num_cores=NC, num_subcores=NS)
_SC_PARAMS = pltpu.CompilerParams(use_tc_tiling_on_sc=False,
                                  needs_layout_passes=False)
_sc_kernel = functools.partial(pl.kernel, mesh=_mesh,
                               compiler_params=_SC_PARAMS)


def _wid():
    return lax.axis_index("s") * NC + lax.axis_index("c")


def _lrelu(x):
    return jnp.maximum(x, 0.01 * x)


def _sc_gather_rows(src, table):
    nblk = EW // GB

    @functools.partial(
        _sc_kernel,
        out_type=jax.ShapeDtypeStruct((EPAD, NF), jnp.float32),
        scratch_types=[
            pltpu.VMEM((GB,), jnp.int32),
            pltpu.VMEM((GB, NF), jnp.float32),
            pltpu.SemaphoreType.DMA,
        ],
    )
    def k(src_hbm, tab_hbm, out_hbm, idx_v, rows_v, sem):
        w = _wid()
        base0 = w * EW

        def blk(b, _):
            base = pl.multiple_of(base0 + b * GB, 8)
            pltpu.sync_copy(src_hbm.at[pl.ds(base, GB)], idx_v)
            pltpu.async_copy(tab_hbm.at[idx_v], rows_v, sem).wait()
            pltpu.sync_copy(rows_v, out_hbm.at[pl.ds(base, GB), :])
            return 0

        lax.fori_loop(0, nblk, blk, 0)

    return k(src, table)


def _sc_attention(u, dst, bias16, t=None, v=None, src=None):
    seq = t is not None
    nblk = EW // EB

    scratch = [
        pltpu.VMEM((NPAD,), jnp.float32),
        pltpu.VMEM((NPAD,), jnp.float32),
        pltpu.VMEM((EB,), jnp.int32),
        pltpu.VMEM((EB,), jnp.int32),
        pltpu.VMEM((EB,), jnp.float32),
        pltpu.VMEM((16,), jnp.float32),
    ]
    out_type = [
        jax.ShapeDtypeStruct((EPAD,), jnp.float32),
        jax.ShapeDtypeStruct((NW, NPAD), jnp.float32),
    ]

    def body(*refs):
        if seq:
            u_hbm, dst_hbm, b_hbm, t_hbm = refs[:4]
            rest = refs[4:]
        else:
            u_hbm, dst_hbm, b_hbm, v_hbm, src_hbm = refs[:5]
            rest = refs[5:]
        e_hbm, s_hbm, u_v, v_v, dst_v, src_v, t_v, b_v = rest
        w = _wid()
        base0 = w * EW
        pltpu.sync_copy(b_hbm, b_v)
        pltpu.sync_copy(u_hbm, u_v)
        if not seq:
            pltpu.sync_copy(v_hbm, v_v)
        b16 = b_v[...]

        def blk1(b, _):
            base = pl.multiple_of(base0 + b * EB, 8)
            pltpu.sync_copy(dst_hbm.at[pl.ds(base, EB)], dst_v)
            if seq:
                pltpu.sync_copy(t_hbm.at[pl.ds(base, EB)], t_v)

                def grp(g, _):
                    d16 = dst_v[pl.ds(g * L, L)]
                    lg = plsc.load_gather(u_v, [d16]) + t_v[pl.ds(g * L, L)] + b16
                    lg = jnp.minimum(_lrelu(lg), CLAMP)
                    t_v[pl.ds(g * L, L)] = jnp.exp(lg)
                    return 0

                lax.fori_loop(0, NEG, grp, 0, unroll=2)
            else:
                pltpu.sync_copy(src_hbm.at[pl.ds(base, EB)], src_v)

                def grp(g, _):
                    d16 = dst_v[pl.ds(g * L, L)]
                    s16 = src_v[pl.ds(g * L, L)]
                    lg = (plsc.load_gather(u_v, [d16])
                          + plsc.load_gather(v_v, [s16]) + b16)
                    lg = jnp.minimum(_lrelu(lg), CLAMP)
                    t_v[pl.ds(g * L, L)] = jnp.exp(lg)
                    return 0

                lax.fori_loop(0, NEG, grp, 0, unroll=2)
            pltpu.sync_copy(t_v, e_hbm.at[pl.ds(base, EB)])
            return 0

        lax.fori_loop(0, nblk, blk1, 0)

        def zero(i, _):
            v_v[pl.ds(i * L, L)] = jnp.zeros((L,), jnp.float32)
            return 0

        lax.fori_loop(0, NPAD // L, zero, 0, unroll=8)

        def blk2(b, _):
            base = pl.multiple_of(base0 + b * EB, 8)
            pltpu.sync_copy(dst_hbm.at[pl.ds(base, EB)], dst_v)
            pltpu.sync_copy(e_hbm.at[pl.ds(base, EB)], t_v)

            def grp(g, _):
                d16 = dst_v[pl.ds(g * L, L)]
                plsc.addupdate_scatter(v_v, [d16], t_v[pl.ds(g * L, L)])
                return 0

            lax.fori_loop(0, NEG, grp, 0, unroll=2)
            return 0

        lax.fori_loop(0, nblk, blk2, 0)
        pltpu.sync_copy(v_v, s_hbm.at[w])

    if seq:
        return _sc_kernel(body, out_type=out_type,
                          scratch_types=scratch)(u, dst, bias16, t)
    return _sc_kernel(body, out_type=out_type,
                      scratch_types=scratch)(u, dst, bias16, v, src)


def _sc_acompute(e, s1d, dst):
    nblk = EW // EB

    @functools.partial(
        _sc_kernel,
        out_type=jax.ShapeDtypeStruct((EPAD,), jnp.float32),
        scratch_types=[
            pltpu.VMEM((NPAD,), jnp.float32),
            pltpu.VMEM((EB,), jnp.int32),
            pltpu.VMEM((EB,), jnp.float32),
        ],
    )
    def k(e_hbm, s_hbm, dst_hbm, a_hbm, s_v, dst_v, e_v):
        w = _wid()
        base0 = w * EW
        pltpu.sync_copy(s_hbm, s_v)

        def blk(b, _):
            base = pl.multiple_of(base0 + b * EB, 8)
            pltpu.sync_copy(dst_hbm.at[pl.ds(base, EB)], dst_v)
            pltpu.sync_copy(e_hbm.at[pl.ds(base, EB)], e_v)

            def grp(g, _):
                d16 = dst_v[pl.ds(g * L, L)]
                sv = plsc.load_gather(s_v, [d16])
                e_v[pl.ds(g * L, L)] = e_v[pl.ds(g * L, L)] / sv
                return 0

            lax.fori_loop(0, NEG, grp, 0, unroll=2)
            pltpu.sync_copy(e_v, a_hbm.at[pl.ds(base, EB)])
            return 0

        lax.fori_loop(0, nblk, blk, 0)

    return k(e, s1d, dst)


def _sc_weighted_scatter(a, dst, rows_src, src=None):
    seq = src is None
    EBL = 1280
    nblk = EW // EBL
    ZR = 784

    scratch = [
        pltpu.VMEM((EBL,), jnp.float32), pltpu.VMEM((EBL,), jnp.float32),
        pltpu.VMEM((EBL,), jnp.int32), pltpu.VMEM((EBL,), jnp.int32),
        pltpu.VMEM((EBL,), jnp.int32), pltpu.VMEM((EBL,), jnp.int32),
        pltpu.VMEM((EBL, L), jnp.float32), pltpu.VMEM((EBL, L), jnp.float32),
        pltpu.VMEM((ZR, L), jnp.float32),
        pltpu.VMEM_SHARED((NPAD, L), jnp.float32),
        pltpu.SemaphoreType.DMA, pltpu.SemaphoreType.DMA,
        pltpu.SemaphoreType.DMA, pltpu.SemaphoreType.DMA,
    ]

    def body(*refs):
        if seq:
            a_hbm, dst_hbm, h_hbm = refs[:3]
            rest = refs[3:]
        else:
            a_hbm, dst_hbm, h_hbm, src_hbm = refs[:4]
            rest = refs[4:]
        (c_hbm, a0, a1, d0, d1, s0, s1, r0, r1, zz_v, acc,
         ig0, ig1, gs0, gs1) = rest
        AV, DV, SV, RV = [a0, a1], [d0, d1], [s0, s1], [r0, r1]
        IS, GS = [ig0, ig1], [gs0, gs1]
        cid = lax.axis_index("c")
        sid = lax.axis_index("s")
        w = sid * NC + cid
        base0 = w * EW
        rb = sid * SROW

        def zzero(i, _):
            zz_v[i, :] = jnp.zeros((L,), jnp.float32)
            return 0

        lax.fori_loop(0, ZR, zzero, 0, unroll=8)

        def issue_ids(b, p):
            base = pl.multiple_of(base0 + b * EBL, 8)
            pltpu.async_copy(a_hbm.at[pl.ds(base, EBL)], AV[p], IS[p])
            pltpu.async_copy(dst_hbm.at[pl.ds(base, EBL)], DV[p], IS[p])
            if not seq:
                pltpu.async_copy(src_hbm.at[pl.ds(base, EBL)], SV[p], IS[p])

        def wait_ids(p):
            pltpu.make_async_copy(a_hbm.at[pl.ds(0, EBL)], AV[p], IS[p]).wait()
            pltpu.make_async_copy(dst_hbm.at[pl.ds(0, EBL)], DV[p],
                                  IS[p]).wait()
            if not seq:
                pltpu.make_async_copy(src_hbm.at[pl.ds(0, EBL)], SV[p],
                                      IS[p]).wait()

        def issue_rows(b, p, f):
            if seq:
                base = pl.multiple_of(base0 + b * EBL, 8)
                pltpu.async_copy(
                    h_hbm.at[pl.ds(base, EBL),
                             pl.ds(pl.multiple_of(f * L, 8), L)], RV[p], GS[p])
            else:
                pltpu.async_copy(h_hbm.at[f].at[SV[p]], RV[p], GS[p])

        def wait_rows(p):
            if seq:
                pltpu.make_async_copy(
                    h_hbm.at[pl.ds(0, EBL), pl.ds(0, L)], RV[p], GS[p]).wait()
            else:
                pltpu.make_async_copy(
                    h_hbm.at[0, pl.ds(0, EBL), :], RV[p], GS[p]).wait()

        def chunk(f, _):
            f16 = pl.multiple_of(f * L, 8)
            for z in range(SROW // ZR):
                pltpu.sync_copy(zz_v, acc.at[pl.ds(rb + z * ZR, ZR), :])
            plsc.subcore_barrier()

            issue_ids(0, 0)
            wait_ids(0)
            issue_rows(0, 0, f)
            issue_ids(1, 1)

            def pair(kk, _):
                for p in range(2):
                    b = 2 * kk + p
                    @pl.when(b + 1 < nblk)
                    def _():
                        wait_ids(1 - p)
                        issue_rows(b + 1, 1 - p, f)

                    wait_rows(p)
                    iota = lax.iota(jnp.int32, L)

                    def scale(g, _):
                        rowi = g * L + iota
                        a16 = AV[p][pl.ds(g * L, L)]
                        for q in range(L):
                            colq = jnp.full((L,), q, jnp.int32)
                            col = plsc.load_gather(RV[p], [rowi, colq])
                            plsc.store_scatter(RV[p], [rowi, colq], col * a16)
                        return 0

                    lax.fori_loop(0, EBL // L, scale, 0, unroll=2)
                    pltpu.sync_copy(RV[p], acc.at[DV[p]], add=True)

                    @pl.when(b + 2 < nblk)
                    def _():
                        issue_ids(b + 2, p)
                return 0

            lax.fori_loop(0, nblk // 2, pair, 0)
            plsc.subcore_barrier()
            pltpu.sync_copy(acc.at[pl.ds(rb, SROW), :],
                            c_hbm.at[cid, pl.ds(rb, SROW), pl.ds(f16, L)])
            plsc.subcore_barrier()
            return 0

        lax.fori_loop(0, NCHUNK, chunk, 0)

    out_type = jax.ShapeDtypeStruct((NC, NPAD, GF), jnp.float32)
    if seq:
        return _sc_kernel(body, out_type=out_type,
                          scratch_types=scratch)(a, dst, rows_src)
    return _sc_kernel(body, out_type=out_type,
                      scratch_types=scratch)(a, dst, rows_src, src)


def _sc_chunkify(h):
    nblk = NB // NBB

    @functools.partial(
        _sc_kernel,
        out_type=jax.ShapeDtypeStruct((NCHUNK, NPAD, L), jnp.float32),
        scratch_types=[
            pltpu.VMEM((NBB, GF), jnp.float32),
            pltpu.VMEM((NBB, L), jnp.float32),
        ],
    )
    def k(h_hbm, hT_hbm, slab_v, tmp_v):
        w = _wid()
        base0 = w * NB

        def blk(b, _):
            base = pl.multiple_of(base0 + b * NBB, 8)
            pltpu.sync_copy(h_hbm.at[pl.ds(base, NBB), :], slab_v)
            for f in range(NCHUNK):
                def mv(j, _):
                    tmp_v[j, :] = slab_v[j, pl.ds(f * L, L)]
                    return 0

                lax.fori_loop(0, NBB, mv, 0, unroll=4)
                pltpu.sync_copy(tmp_v, hT_hbm.at[f, pl.ds(base, NBB), :])
            return 0

        lax.fori_loop(0, nblk, blk, 0)

    return k(h)


def _sc_segsum_rows(h, ids, weights=None):
    nblk = NB // NBB

    scratch = [
        pltpu.VMEM((NBB, GF), jnp.float32),
        pltpu.VMEM((NBB,), jnp.int32),
        pltpu.VMEM((NBB,), jnp.float32),
        pltpu.VMEM((GROW, GF), jnp.float32),
        pltpu.VMEM_SHARED((GA, GF), jnp.float32),
    ]

    def body(*refs):
        if weights is None:
            h_hbm, ids_hbm = refs[:2]
            rest = refs[2:]
        else:
            h_hbm, ids_hbm, w_hbm = refs[:3]
            rest = refs[3:]
        g_hbm, rows_v, idx_v, wt_v, zz_v, acc = rest
        cid = lax.axis_index("c")
        sid = lax.axis_index("s")
        w = sid * NC + cid
        base0 = w * NB
        rb = sid * GROW

        def zzero(i, _):
            for q in range(NCHUNK):
                zz_v[i, pl.ds(q * L, L)] = jnp.zeros((L,), jnp.float32)
            return 0

        lax.fori_loop(0, GROW, zzero, 0, unroll=4)
        pltpu.sync_copy(zz_v, acc.at[pl.ds(rb, GROW), :])
        plsc.subcore_barrier()

        def blk(b, _):
            base = pl.multiple_of(base0 + b * NBB, 8)
            pltpu.sync_copy(h_hbm.at[pl.ds(base, NBB), :], rows_v)
            pltpu.sync_copy(ids_hbm.at[pl.ds(base, NBB)], idx_v)
            if weights is not None:
                pltpu.sync_copy(w_hbm.at[pl.ds(base, NBB)], wt_v)

                def scale(j, _):
                    av = plsc.load_gather(wt_v, [jnp.full((L,), j, jnp.int32)])
                    for q in range(NCHUNK):
                        rows_v[j, pl.ds(q * L, L)] = (
                            rows_v[j, pl.ds(q * L, L)] * av)
                    return 0

                lax.fori_loop(0, NBB, scale, 0, unroll=2)
            pltpu.sync_copy(rows_v, acc.at[idx_v], add=True)
            return 0

        lax.fori_loop(0, nblk, blk, 0)
        plsc.subcore_barrier()
        pltpu.sync_copy(acc.at[pl.ds(rb, GROW), :],
                        g_hbm.at[cid, pl.ds(rb, GROW), :])

    out_type = jax.ShapeDtypeStruct((NC, GA, GF), jnp.float32)
    if weights is None:
        return _sc_kernel(body, out_type=out_type,
                          scratch_types=scratch)(h, ids)
    return _sc_kernel(body, out_type=out_type,
                      scratch_types=scratch)(h, ids, weights)


def _sc_attention_ro(alpha, beta, ids, bias16):
    nblk = NB // NBB

    @functools.partial(
        _sc_kernel,
        out_type=[
            jax.ShapeDtypeStruct((NPAD,), jnp.float32),
            jax.ShapeDtypeStruct((NW, GA), jnp.float32),
        ],
        scratch_types=[
            pltpu.VMEM((GA,), jnp.float32),
            pltpu.VMEM((GA,), jnp.float32),
            pltpu.VMEM((NBB,), jnp.int32),
            pltpu.VMEM((NBB,), jnp.float32),
            pltpu.VMEM((16,), jnp.float32),
        ],
    )
    def k(al_hbm, be_hbm, ids_hbm, b_hbm, e_hbm, s_hbm,
          al_v, s_v, idx_v, be_v, b_v):
        w = _wid()
        base0 = w * NB
        pltpu.sync_copy(b_hbm, b_v)
        pltpu.sync_copy(al_hbm, al_v)
        b16 = b_v[...]

        def zero(i, _):
            s_v[pl.ds(i * L, L)] = jnp.zeros((L,), jnp.float32)
            return 0

        lax.fori_loop(0, GA // L, zero, 0, unroll=8)

        def blk(b, _):
            base = pl.multiple_of(base0 + b * NBB, 8)
            pltpu.sync_copy(ids_hbm.at[pl.ds(base, NBB)], idx_v)
            pltpu.sync_copy(be_hbm.at[pl.ds(base, NBB)], be_v)

            def grp(g, _):
                d16 = idx_v[pl.ds(g * L, L)]
                lg = plsc.load_gather(al_v, [d16]) + be_v[pl.ds(g * L, L)] + b16
                lg = jnp.minimum(_lrelu(lg), CLAMP)
                ev = jnp.exp(lg)
                be_v[pl.ds(g * L, L)] = ev
                plsc.addupdate_scatter(s_v, [d16], ev)
                return 0

            lax.fori_loop(0, NBB // L, grp, 0, unroll=2)
            pltpu.sync_copy(be_v, e_hbm.at[pl.ds(base, NBB)])
            return 0

        lax.fori_loop(0, nblk, blk, 0)
        pltpu.sync_copy(s_v, s_hbm.at[w])

    return k(alpha, beta, ids, bias16)


def _sc_acompute_ro(e, s1d, ids):
    nblk = NB // NBB

    @functools.partial(
        _sc_kernel,
        out_type=jax.ShapeDtypeStruct((NPAD,), jnp.float32),
        scratch_types=[
            pltpu.VMEM((GA,), jnp.float32),
            pltpu.VMEM((NBB,), jnp.int32),
            pltpu.VMEM((NBB,), jnp.float32),
        ],
    )
    def k(e_hbm, s_hbm, ids_hbm, a_hbm, s_v, idx_v, e_v):
        w = _wid()
        base0 = w * NB
        pltpu.sync_copy(s_hbm, s_v)

        def blk(b, _):
            base = pl.multiple_of(base0 + b * NBB, 8)
            pltpu.sync_copy(ids_hbm.at[pl.ds(base, NBB)], idx_v)
            pltpu.sync_copy(e_hbm.at[pl.ds(base, NBB)], e_v)

            def grp(g, _):
                d16 = idx_v[pl.ds(g * L, L)]
                sv = plsc.load_gather(s_v, [d16])
                e_v[pl.ds(g * L, L)] = e_v[pl.ds(g * L, L)] / sv
                return 0

            lax.fori_loop(0, NBB // L, grp, 0, unroll=2)
            pltpu.sync_copy(e_v, a_hbm.at[pl.ds(base, NBB)])
            return 0

        lax.fori_loop(0, nblk, blk, 0)

    return k(e, s1d, ids)


def _tc_node0(nf, WT, b, w1):
    BLK = 1024

    def body(nf_ref, wt_ref, b_ref, w1_ref, hv_ref, u_ref):
        y = jnp.dot(nf_ref[...], wt_ref[...],
                    preferred_element_type=jnp.float32) + b_ref[...]
        hv = _lrelu(y)
        hv_ref[...] = hv
        u_ref[...] = jnp.sum(hv * w1_ref[...], axis=1, keepdims=True)

    return pl.pallas_call(
        body,
        grid=(NPAD // BLK,),
        in_specs=[
            pl.BlockSpec((BLK, NF), lambda i: (i, 0)),
            pl.BlockSpec((NF, GF), lambda i: (0, 0)),
            pl.BlockSpec((1, GF), lambda i: (0, 0)),
            pl.BlockSpec((1, GF), lambda i: (0, 0)),
        ],
        out_specs=[
            pl.BlockSpec((BLK, GF), lambda i: (i, 0)),
            pl.BlockSpec((BLK, 1), lambda i: (i, 0)),
        ],
        out_shape=[
            jax.ShapeDtypeStruct((NPAD, GF), jnp.float32),
            jax.ShapeDtypeStruct((NPAD, 1), jnp.float32),
        ],
    )(nf, WT, b, w1)


def _tc_he1(nfs, ef, WT, b, w2):
    BLK = 2048

    def body(nfs_ref, ef_ref, wt_ref, b_ref, w2_ref, he_ref, t_ref):
        x = jnp.concatenate([nfs_ref[...], ef_ref[...]], axis=1)
        he1 = _lrelu(jnp.dot(x, wt_ref[...],
                             preferred_element_type=jnp.float32) + b_ref[...])
        he_ref[...] = he1
        t_ref[...] = jnp.sum(he1 * w2_ref[...], axis=1, keepdims=True)

    return pl.pallas_call(
        body,
        grid=(EPAD // BLK,),
        in_specs=[
            pl.BlockSpec((BLK, NF), lambda i: (i, 0)),
            pl.BlockSpec((BLK, EF), lambda i: (i, 0)),
            pl.BlockSpec((NF + EF, GF), lambda i: (0, 0)),
            pl.BlockSpec((1, GF), lambda i: (0, 0)),
            pl.BlockSpec((1, GF), lambda i: (0, 0)),
        ],
        out_specs=[
            pl.BlockSpec((BLK, GF), lambda i: (i, 0)),
            pl.BlockSpec((BLK, 1), lambda i: (i, 0)),
        ],
        out_shape=[
            jax.ShapeDtypeStruct((EPAD, GF), jnp.float32),
            jax.ShapeDtypeStruct((EPAD, 1), jnp.float32),
        ],
    )(nfs, ef, WT, b, w2)


def _tc_sumcols(s_all, width):
    BLK = 6272 if width == NPAD else width
    grid = width // BLK

    def body(s_ref, o_ref):
        o_ref[...] = jnp.sum(s_ref[...], axis=0, keepdims=True)

    return pl.pallas_call(
        body,
        grid=(grid,),
        in_specs=[pl.BlockSpec((NW, BLK), lambda i: (0, i))],
        out_specs=pl.BlockSpec((1, BLK), lambda i: (0, i)),
        out_shape=jax.ShapeDtypeStruct((1, width), jnp.float32),
    )(s_all)


def _gru_math(x, h, WihT, bih, WhhT, bhh):
    gi = jnp.dot(x, WihT, preferred_element_type=jnp.float32) + bih
    gh = jnp.dot(h, WhhT, preferred_element_type=jnp.float32) + bhh
    i_r, i_z, i_n = gi[:, :GF], gi[:, GF:2 * GF], gi[:, 2 * GF:]
    h_r, h_z, h_n = gh[:, :GF], gh[:, GF:2 * GF], gh[:, 2 * GF:]
    r = jax.nn.sigmoid(i_r + h_r)
    z = jax.nn.sigmoid(i_z + h_z)
    n = jnp.tanh(i_n + r * h_n)
    return (1.0 - z) * n + z * h


def _elu(c):
    return jnp.where(c > 0, c, jnp.exp(jnp.minimum(c, 0.0)) - 1.0)


def _tc_gru(cpart, h, pnWT, pnb, WihT, bih, WhhT, bhh, w1, w2):
    BLK = 1024

    def body(c_ref, h_ref, pnwt_ref, pnb_ref, wih_ref, bih_ref,
             whh_ref, bhh_ref, w1_ref, w2_ref, hn_ref, u_ref, v_ref):
        c = jnp.dot(c_ref[0] + c_ref[1], pnwt_ref[...],
                    preferred_element_type=jnp.float32) + pnb_ref[...]
        hn = jnp.maximum(_gru_math(_elu(c), h_ref[...], wih_ref[...],
                                   bih_ref[...], whh_ref[...], bhh_ref[...]),
                         0.0)
        hn_ref[...] = hn
        u_ref[...] = jnp.sum(hn * w1_ref[...], axis=1, keepdims=True)
        v_ref[...] = jnp.sum(hn * w2_ref[...], axis=1, keepdims=True)

    return pl.pallas_call(
        body,
        grid=(NPAD // BLK,),
        in_specs=[
            pl.BlockSpec((NC, BLK, GF), lambda i: (0, i, 0)),
            pl.BlockSpec((BLK, GF), lambda i: (i, 0)),
            pl.BlockSpec((GF, GF), lambda i: (0, 0)),
            pl.BlockSpec((1, GF), lambda i: (0, 0)),
            pl.BlockSpec((GF, 3 * GF), lambda i: (0, 0)),
            pl.BlockSpec((1, 3 * GF), lambda i: (0, 0)),
            pl.BlockSpec((GF, 3 * GF), lambda i: (0, 0)),
            pl.BlockSpec((1, 3 * GF), lambda i: (0, 0)),
            pl.BlockSpec((1, GF), lambda i: (0, 0)),
            pl.BlockSpec((1, GF), lambda i: (0, 0)),
        ],
        out_specs=[
            pl.BlockSpec((BLK, GF), lambda i: (i, 0)),
            pl.BlockSpec((BLK, 1), lambda i: (i, 0)),
            pl.BlockSpec((BLK, 1), lambda i: (i, 0)),
        ],
        out_shape=[
            jax.ShapeDtypeStruct((NPAD, GF), jnp.float32),
            jax.ShapeDtypeStruct((NPAD, 1), jnp.float32),
            jax.ShapeDtypeStruct((NPAD, 1), jnp.float32),
        ],
    )(cpart, h, pnWT, pnb, WihT, bih, WhhT, bhh, w1, w2)


def _tc_r1(g0, g1, w1):
    def body(g0_ref, g1_ref, w1_ref, gf_ref, al_ref):
        gf = g0_ref[...] + g1_ref[...]
        gf_ref[...] = gf
        al_ref[...] = jnp.sum(jnp.maximum(gf, 0.0) * w1_ref[...],
                              axis=1, keepdims=True)

    return pl.pallas_call(
        body,
        out_shape=[
            jax.ShapeDtypeStruct((G, GF), jnp.float32),
            jax.ShapeDtypeStruct((G, 1), jnp.float32),
        ],
    )(g0, g1, w1)


def _tc_gru_g(gr0, gr1, gf, pnWT, pnb, WihT, bih, WhhT, bhh):
    def body(g0_ref, g1_ref, gf_ref, pnwt_ref, pnb_ref, wih_ref, bih_ref,
             whh_ref, bhh_ref, o_ref):
        c = jnp.dot(g0_ref[...] + g1_ref[...], pnwt_ref[...],
                    preferred_element_type=jnp.float32) + pnb_ref[...]
        o_ref[...] = jnp.maximum(
            _gru_math(_elu(c), gf_ref[...], wih_ref[...], bih_ref[...],
                      whh_ref[...], bhh_ref[...]), 0.0)

    return pl.pallas_call(
        body,
        out_shape=jax.ShapeDtypeStruct((G, GF), jnp.float32),
    )(gr0, gr1, gf, pnWT, pnb, WihT, bih, WhhT, bhh)


def _tc_head(fpr, fp1T, fp1b, fp2T, fp2b, g2, pr1T, pr1b, pr2row, pr2bb):
    def body(fp_ref, w1_ref, b1_ref, w2_ref, b2_ref, g2_ref, p1_ref, pb1_ref,
             p2_ref, pb2_ref, o_ref):
        fph = jnp.maximum(jnp.dot(fp_ref[...], w1_ref[...],
                                  preferred_element_type=jnp.float32)
                          + b1_ref[...], 0.0)
        fpf = jnp.dot(fph, w2_ref[...],
                      preferred_element_type=jnp.float32) + b2_ref[...]
        comb = jnp.concatenate([g2_ref[...], fpf], axis=1)
        hh = jnp.maximum(jnp.dot(comb, p1_ref[...],
                                 preferred_element_type=jnp.float32)
                         + pb1_ref[...], 0.0)
        yy = hh * p2_ref[...] + pb2_ref[...]
        o_ref[...] = jnp.sum(yy, axis=1, keepdims=True)

    return pl.pallas_call(
        body,
        out_shape=jax.ShapeDtypeStruct((G, 1), jnp.float32),
    )(fpr, fp1T, fp1b, fp2T, fp2b, g2, pr1T, pr1b, pr2row, pr2bb)


def kernel(node_feats, edge_feats, fingerprints, params, edge_index, node2graph):
    p = params
    f32 = jnp.float32
    i32 = jnp.int32

    nf_pad = jnp.pad(node_feats, ((0, NPAD - N), (0, 0)))
    src = jnp.concatenate([edge_index[0], jnp.zeros((EPAD - E,), i32)])
    dst = jnp.concatenate([edge_index[1],
                           jnp.full((EPAD - E,), DUMMY_DST, i32)])
    ef_pad = jnp.pad(edge_feats, ((0, EPAD - E), (0, 0)))
    n2g = jnp.concatenate([node2graph, jnp.full((NPAD - N,), DUMMY_G, i32)])

    row = lambda x: x.reshape(1, -1)
    bvec = lambda s: jnp.broadcast_to(s.reshape(()), (16,)).astype(f32)

    gc_pe2_w1 = row(p['gc_pe2_W'][0, :GF])
    gc_pe2_w2 = row(p['gc_pe2_W'][0, GF:])
    l0_w1 = row(p['l0_pe_W'][0, :GF])
    l0_w2 = row(p['l0_pe_W'][0, GF:])
    l1_w1 = row(p['l1_pe_W'][0, :GF])
    l1_w2 = row(p['l1_pe_W'][0, GF:])
    cl_w1 = row(p['ro0_cl_W'][0, :GF])
    cl_w2 = row(p['ro0_cl_W'][0, GF:])

    hv, u0 = _tc_node0(nf_pad, p['gc_pn_W'].T, row(p['gc_pn_b']), gc_pe2_w1)
    nfs = _sc_gather_rows(src, node_feats)
    he1, t = _tc_he1(nfs, ef_pad, p['gc_pe1_W'].T, row(p['gc_pe1_b']),
                     gc_pe2_w2)
    e_gc, sall_gc = _sc_attention(u0.reshape(NPAD), dst, bvec(p['gc_pe2_b']),
                                  t=t.reshape(EPAD))
    s_gc = _tc_sumcols(sall_gc, NPAD).reshape(NPAD)
    a_gc = _sc_acompute(e_gc, s_gc, dst)
    c_gc = _sc_weighted_scatter(a_gc, dst, he1)
    h1, u1, v1 = _tc_gru(c_gc, hv, p['gc_et_W'].T, row(p['gc_et_b']),
                         p['gc_Wih'].T, row(p['gc_bih']), p['gc_Whh'].T,
                         row(p['gc_bhh']), l0_w1, l0_w2)

    hT1 = _sc_chunkify(h1)
    e0, sall0 = _sc_attention(u1.reshape(NPAD), dst, bvec(p['l0_pe_b']),
                              v=v1.reshape(NPAD), src=src)
    s0 = _tc_sumcols(sall0, NPAD).reshape(NPAD)
    a0 = _sc_acompute(e0, s0, dst)
    c0 = _sc_weighted_scatter(a0, dst, hT1, src=src)
    h2, u2, v2 = _tc_gru(c0, h1, p['l0_pn_W'].T, row(p['l0_pn_b']),
                         p['l0_Wih'].T, row(p['l0_bih']), p['l0_Whh'].T,
                         row(p['l0_bhh']), l1_w1, l1_w2)

    hT2 = _sc_chunkify(h2)
    e1, sall1 = _sc_attention(u2.reshape(NPAD), dst, bvec(p['l1_pe_b']),
                              v=v2.reshape(NPAD), src=src)
    s1 = _tc_sumcols(sall1, NPAD).reshape(NPAD)
    a1 = _sc_acompute(e1, s1, dst)
    c1 = _sc_weighted_scatter(a1, dst, hT2, src=src)
    h3, beta, _v3 = _tc_gru(c1, h2, p['l1_pn_W'].T, row(p['l1_pn_b']),
                            p['l1_Wih'].T, row(p['l1_bih']), p['l1_Whh'].T,
                            row(p['l1_bhh']), cl_w2, cl_w2)

    gt = _sc_segsum_rows(h3, n2g)
    gf, alpha = _tc_r1(gt[0, :G], gt[1, :G], cl_w1)
    alpha_pad = jnp.pad(alpha.reshape(G), (0, GA - G))
    e_ro, sall_ro = _sc_attention_ro(alpha_pad, beta.reshape(NPAD), n2g,
                                    bvec(p['ro0_cl_b']))
    s_ro = _tc_sumcols(sall_ro, GA).reshape(GA)
    an = _sc_acompute_ro(e_ro, s_ro, n2g)
    gr = _sc_segsum_rows(h3, n2g, weights=an)
    g2 = _tc_gru_g(gr[0, :G], gr[1, :G], gf, p['ro0_pn_W'].T,
                   row(p['ro0_pn_b']), p['ro0_Wih'].T, row(p['ro0_bih']),
                   p['ro0_Whh'].T, row(p['ro0_bhh']))

    pr2bb = jnp.broadcast_to(p['pr2_b'].reshape(()) / 128.0, (1, 128)).astype(f32)
    out = _tc_head(fingerprints, p['fp1_W'].T, row(p['fp1_b']), p['fp2_W'].T,
                   row(p['fp2_b']), g2, p['pr1_W'].T, row(p['pr1_b']),
                   row(p['pr2_W'][0]), pr2bb)
    return out


# speedup vs baseline: 1.2808x; 1.2808x over previous
import functools

import jax
import jax.numpy as jnp
from jax import lax
from jax.experimental import pallas as pl
from jax.experimental.pallas import tpu as pltpu
from jax.experimental.pallas import tpu_sc as plsc

N = 50000
E = 800000
G = 1024
NF = 64
EF = 16
GF = 256
FP = 4096

NC = 2
NS = 16
NW = NC * NS
L = 16

NPAD = 50176
EPAD = 819200
EW = EPAD // NW
EB = 2560
NEG = EB // L
GB = 640
SROW = NPAD // NS
GA = 1152
GROW = GA // NS
NB = NPAD // NW
NBB = 224
DUMMY_DST = 50100
DUMMY_G = 1024
CLAMP = 45.0
NCHUNK = GF // L

_mesh = plsc.VectorSubcoreMesh(core_axis_name="c", subcore_axis_name="s",
                               num_cores=NC, num_subcores=NS)
_SC_PARAMS = pltpu.CompilerParams(use_tc_tiling_on_sc=False,
                                  needs_layout_passes=False)
_sc_kernel = functools.partial(pl.kernel, mesh=_mesh,
                               compiler_params=_SC_PARAMS)


def _wid():
    return lax.axis_index("s") * NC + lax.axis_index("c")


def _lrelu(x):
    return jnp.maximum(x, 0.01 * x)


def _sc_gather_rows(src, table):
    nblk = EW // GB

    @functools.partial(
        _sc_kernel,
        out_type=jax.ShapeDtypeStruct((EPAD, NF), jnp.float32),
        scratch_types=[
            pltpu.VMEM((GB,), jnp.int32),
            pltpu.VMEM((GB, NF), jnp.float32),
            pltpu.SemaphoreType.DMA,
        ],
    )
    def k(src_hbm, tab_hbm, out_hbm, idx_v, rows_v, sem):
        w = _wid()
        base0 = w * EW

        def blk(b, _):
            base = pl.multiple_of(base0 + b * GB, 8)
            pltpu.sync_copy(src_hbm.at[pl.ds(base, GB)], idx_v)
            pltpu.async_copy(tab_hbm.at[idx_v], rows_v, sem).wait()
            pltpu.sync_copy(rows_v, out_hbm.at[pl.ds(base, GB), :])
            return 0

        lax.fori_loop(0, nblk, blk, 0)

    return k(src, table)


def _sc_attention(u, dst, bias16, t=None, v=None, src=None):
    seq = t is not None
    nblk = EW // EB

    scratch = [
        pltpu.VMEM((NPAD,), jnp.float32),
        pltpu.VMEM((NPAD,), jnp.float32),
        pltpu.VMEM((EB,), jnp.int32),
        pltpu.VMEM((EB,), jnp.int32),
        pltpu.VMEM((EB,), jnp.float32),
        pltpu.VMEM((16,), jnp.float32),
    ]
    out_type = [
        jax.ShapeDtypeStruct((EPAD,), jnp.float32),
        jax.ShapeDtypeStruct((NW, NPAD), jnp.float32),
    ]

    def body(*refs):
        if seq:
            u_hbm, dst_hbm, b_hbm, t_hbm = refs[:4]
            rest = refs[4:]
        else:
            u_hbm, dst_hbm, b_hbm, v_hbm, src_hbm = refs[:5]
            rest = refs[5:]
        e_hbm, s_hbm, u_v, v_v, dst_v, src_v, t_v, b_v = rest
        w = _wid()
        base0 = w * EW
        pltpu.sync_copy(b_hbm, b_v)
        pltpu.sync_copy(u_hbm, u_v)
        if not seq:
            pltpu.sync_copy(v_hbm, v_v)
        b16 = b_v[...]

        def blk1(b, _):
            base = pl.multiple_of(base0 + b * EB, 8)
            pltpu.sync_copy(dst_hbm.at[pl.ds(base, EB)], dst_v)
            if seq:
                pltpu.sync_copy(t_hbm.at[pl.ds(base, EB)], t_v)

                def grp(g, _):
                    d16 = dst_v[pl.ds(g * L, L)]
                    lg = plsc.load_gather(u_v, [d16]) + t_v[pl.ds(g * L, L)] + b16
                    lg = jnp.minimum(_lrelu(lg), CLAMP)
                    t_v[pl.ds(g * L, L)] = jnp.exp(lg)
                    return 0

                lax.fori_loop(0, NEG, grp, 0, unroll=2)
            else:
                pltpu.sync_copy(src_hbm.at[pl.ds(base, EB)], src_v)

                def grp(g, _):
                    d16 = dst_v[pl.ds(g * L, L)]
                    s16 = src_v[pl.ds(g * L, L)]
                    lg = (plsc.load_gather(u_v, [d16])
                          + plsc.load_gather(v_v, [s16]) + b16)
                    lg = jnp.minimum(_lrelu(lg), CLAMP)
                    t_v[pl.ds(g * L, L)] = jnp.exp(lg)
                    return 0

                lax.fori_loop(0, NEG, grp, 0, unroll=2)
            pltpu.sync_copy(t_v, e_hbm.at[pl.ds(base, EB)])
            return 0

        lax.fori_loop(0, nblk, blk1, 0)

        def zero(i, _):
            v_v[pl.ds(i * L, L)] = jnp.zeros((L,), jnp.float32)
            return 0

        lax.fori_loop(0, NPAD // L, zero, 0, unroll=8)

        def blk2(b, _):
            base = pl.multiple_of(base0 + b * EB, 8)
            pltpu.sync_copy(dst_hbm.at[pl.ds(base, EB)], dst_v)
            pltpu.sync_copy(e_hbm.at[pl.ds(base, EB)], t_v)

            def grp(g, _):
                d16 = dst_v[pl.ds(g * L, L)]
                plsc.addupdate_scatter(v_v, [d16], t_v[pl.ds(g * L, L)])
                return 0

            lax.fori_loop(0, NEG, grp, 0, unroll=2)
            return 0

        lax.fori_loop(0, nblk, blk2, 0)
        pltpu.sync_copy(v_v, s_hbm.at[w])

    if seq:
        return _sc_kernel(body, out_type=out_type,
                          scratch_types=scratch)(u, dst, bias16, t)
    return _sc_kernel(body, out_type=out_type,
                      scratch_types=scratch)(u, dst, bias16, v, src)


def _sc_acompute(e, s1d, dst):
    nblk = EW // EB

    @functools.partial(
        _sc_kernel,
        out_type=jax.ShapeDtypeStruct((EPAD,), jnp.float32),
        scratch_types=[
            pltpu.VMEM((NPAD,), jnp.float32),
            pltpu.VMEM((EB,), jnp.int32),
            pltpu.VMEM((EB,), jnp.float32),
        ],
    )
    def k(e_hbm, s_hbm, dst_hbm, a_hbm, s_v, dst_v, e_v):
        w = _wid()
        base0 = w * EW
        pltpu.sync_copy(s_hbm, s_v)

        def blk(b, _):
            base = pl.multiple_of(base0 + b * EB, 8)
            pltpu.sync_copy(dst_hbm.at[pl.ds(base, EB)], dst_v)
            pltpu.sync_copy(e_hbm.at[pl.ds(base, EB)], e_v)

            def grp(g, _):
                d16 = dst_v[pl.ds(g * L, L)]
                sv = plsc.load_gather(s_v, [d16])
                e_v[pl.ds(g * L, L)] = e_v[pl.ds(g * L, L)] / sv
                return 0

            lax.fori_loop(0, NEG, grp, 0, unroll=2)
            pltpu.sync_copy(e_v, a_hbm.at[pl.ds(base, EB)])
            return 0

        lax.fori_loop(0, nblk, blk, 0)

    return k(e, s1d, dst)


def _sc_weighted_scatter(a, dst, rows_src, src=None):
    seq = src is None
    EBL = 1280
    nblk = EW // EBL
    ZR = 784

    scratch = [
        pltpu.VMEM((EBL,), jnp.float32), pltpu.VMEM((EBL,), jnp.float32),
        pltpu.VMEM((EBL,), jnp.int32), pltpu.VMEM((EBL,), jnp.int32),
        pltpu.VMEM((EBL,), jnp.int32), pltpu.VMEM((EBL,), jnp.int32),
        pltpu.VMEM((EBL, L), jnp.float32), pltpu.VMEM((EBL, L), jnp.float32),
        pltpu.VMEM((ZR, L), jnp.float32),
        pltpu.VMEM_SHARED((NPAD, L), jnp.float32),
        pltpu.SemaphoreType.DMA, pltpu.SemaphoreType.DMA,
        pltpu.SemaphoreType.DMA, pltpu.SemaphoreType.DMA,
    ]

    def body(*refs):
        if seq:
            a_hbm, dst_hbm, h_hbm = refs[:3]
            rest = refs[3:]
        else:
            a_hbm, dst_hbm, h_hbm, src_hbm = refs[:4]
            rest = refs[4:]
        (c_hbm, a0, a1, d0, d1, s0, s1, r0, r1, zz_v, acc,
         ig0, ig1, gs0, gs1) = rest
        AV, DV, SV, RV = [a0, a1], [d0, d1], [s0, s1], [r0, r1]
        IS, GS = [ig0, ig1], [gs0, gs1]
        cid = lax.axis_index("c")
        sid = lax.axis_index("s")
        w = sid * NC + cid
        base0 = w * EW
        rb = sid * SROW

        def zzero(i, _):
            zz_v[i, :] = jnp.zeros((L,), jnp.float32)
            return 0

        lax.fori_loop(0, ZR, zzero, 0, unroll=8)

        def issue_ids(b, p):
            base = pl.multiple_of(base0 + b * EBL, 8)
            pltpu.async_copy(a_hbm.at[pl.ds(base, EBL)], AV[p], IS[p])
            pltpu.async_copy(dst_hbm.at[pl.ds(base, EBL)], DV[p], IS[p])
            if not seq:
                pltpu.async_copy(src_hbm.at[pl.ds(base, EBL)], SV[p], IS[p])

        def wait_ids(p):
            pltpu.make_async_copy(a_hbm.at[pl.ds(0, EBL)], AV[p], IS[p]).wait()
            pltpu.make_async_copy(dst_hbm.at[pl.ds(0, EBL)], DV[p],
                                  IS[p]).wait()
            if not seq:
                pltpu.make_async_copy(src_hbm.at[pl.ds(0, EBL)], SV[p],
                                      IS[p]).wait()

        def issue_rows(b, p, f):
            if seq:
                base = pl.multiple_of(base0 + b * EBL, 8)
                pltpu.async_copy(
                    h_hbm.at[pl.ds(base, EBL),
                             pl.ds(pl.multiple_of(f * L, 8), L)], RV[p], GS[p])
            else:
                pltpu.async_copy(h_hbm.at[f].at[SV[p]], RV[p], GS[p])

        def wait_rows(p):
            if seq:
                pltpu.make_async_copy(
                    h_hbm.at[pl.ds(0, EBL), pl.ds(0, L)], RV[p], GS[p]).wait()
            else:
                pltpu.make_async_copy(
                    h_hbm.at[0, pl.ds(0, EBL), :], RV[p], GS[p]).wait()

        def chunk(f, _):
            f16 = pl.multiple_of(f * L, 8)
            for z in range(SROW // ZR):
                pltpu.sync_copy(zz_v, acc.at[pl.ds(rb + z * ZR, ZR), :])
            plsc.subcore_barrier()

            issue_ids(0, 0)
            wait_ids(0)
            issue_rows(0, 0, f)
            issue_ids(1, 1)

            def pair(kk, _):
                for p in range(2):
                    b = 2 * kk + p
                    @pl.when(b + 1 < nblk)
                    def _():
                        wait_ids(1 - p)
                        issue_rows(b + 1, 1 - p, f)

                    wait_rows(p)

                    def scale(j, _):
                        av = plsc.load_gather(
                            AV[p], [jnp.full((L,), j, jnp.int32)])
                        RV[p][j, :] = RV[p][j, :] * av
                        return 0

                    lax.fori_loop(0, EBL, scale, 0, unroll=8)
                    pltpu.sync_copy(RV[p], acc.at[DV[p]], add=True)

                    @pl.when(b + 2 < nblk)
                    def _():
                        issue_ids(b + 2, p)
                return 0

            lax.fori_loop(0, nblk // 2, pair, 0)
            plsc.subcore_barrier()
            pltpu.sync_copy(acc.at[pl.ds(rb, SROW), :],
                            c_hbm.at[cid, pl.ds(rb, SROW), pl.ds(f16, L)])
            plsc.subcore_barrier()
            return 0

        lax.fori_loop(0, NCHUNK, chunk, 0)

    out_type = jax.ShapeDtypeStruct((NC, NPAD, GF), jnp.float32)
    if seq:
        return _sc_kernel(body, out_type=out_type,
                          scratch_types=scratch)(a, dst, rows_src)
    return _sc_kernel(body, out_type=out_type,
                      scratch_types=scratch)(a, dst, rows_src, src)


def _sc_chunkify(h):
    nblk = NB // NBB

    @functools.partial(
        _sc_kernel,
        out_type=jax.ShapeDtypeStruct((NCHUNK, NPAD, L), jnp.float32),
        scratch_types=[
            pltpu.VMEM((NBB, GF), jnp.float32),
            pltpu.VMEM((NBB, L), jnp.float32),
        ],
    )
    def k(h_hbm, hT_hbm, slab_v, tmp_v):
        w = _wid()
        base0 = w * NB

        def blk(b, _):
            base = pl.multiple_of(base0 + b * NBB, 8)
            pltpu.sync_copy(h_hbm.at[pl.ds(base, NBB), :], slab_v)
            for f in range(NCHUNK):
                def mv(j, _):
                    tmp_v[j, :] = slab_v[j, pl.ds(f * L, L)]
                    return 0

                lax.fori_loop(0, NBB, mv, 0, unroll=4)
                pltpu.sync_copy(tmp_v, hT_hbm.at[f, pl.ds(base, NBB), :])
            return 0

        lax.fori_loop(0, nblk, blk, 0)

    return k(h)


def _sc_segsum_rows(h, ids, weights=None):
    nblk = NB // NBB

    scratch = [
        pltpu.VMEM((NBB, GF), jnp.float32),
        pltpu.VMEM((NBB,), jnp.int32),
        pltpu.VMEM((NBB,), jnp.float32),
        pltpu.VMEM((GROW, GF), jnp.float32),
        pltpu.VMEM_SHARED((GA, GF), jnp.float32),
    ]

    def body(*refs):
        if weights is None:
            h_hbm, ids_hbm = refs[:2]
            rest = refs[2:]
        else:
            h_hbm, ids_hbm, w_hbm = refs[:3]
            rest = refs[3:]
        g_hbm, rows_v, idx_v, wt_v, zz_v, acc = rest
        cid = lax.axis_index("c")
        sid = lax.axis_index("s")
        w = sid * NC + cid
        base0 = w * NB
        rb = sid * GROW

        def zzero(i, _):
            for q in range(NCHUNK):
                zz_v[i, pl.ds(q * L, L)] = jnp.zeros((L,), jnp.float32)
            return 0

        lax.fori_loop(0, GROW, zzero, 0, unroll=4)
        pltpu.sync_copy(zz_v, acc.at[pl.ds(rb, GROW), :])
        plsc.subcore_barrier()

        def blk(b, _):
            base = pl.multiple_of(base0 + b * NBB, 8)
            pltpu.sync_copy(h_hbm.at[pl.ds(base, NBB), :], rows_v)
            pltpu.sync_copy(ids_hbm.at[pl.ds(base, NBB)], idx_v)
            if weights is not None:
                pltpu.sync_copy(w_hbm.at[pl.ds(base, NBB)], wt_v)

                def scale(j, _):
                    av = plsc.load_gather(wt_v, [jnp.full((L,), j, jnp.int32)])
                    for q in range(NCHUNK):
                        rows_v[j, pl.ds(q * L, L)] = (
                            rows_v[j, pl.ds(q * L, L)] * av)
                    return 0

                lax.fori_loop(0, NBB, scale, 0, unroll=2)
            pltpu.sync_copy(rows_v, acc.at[idx_v], add=True)
            return 0

        lax.fori_loop(0, nblk, blk, 0)
        plsc.subcore_barrier()
        pltpu.sync_copy(acc.at[pl.ds(rb, GROW), :],
                        g_hbm.at[cid, pl.ds(rb, GROW), :])

    out_type = jax.ShapeDtypeStruct((NC, GA, GF), jnp.float32)
    if weights is None:
        return _sc_kernel(body, out_type=out_type,
                          scratch_types=scratch)(h, ids)
    return _sc_kernel(body, out_type=out_type,
                      scratch_types=scratch)(h, ids, weights)


def _sc_attention_ro(alpha, beta, ids, bias16):
    nblk = NB // NBB

    @functools.partial(
        _sc_kernel,
        out_type=[
            jax.ShapeDtypeStruct((NPAD,), jnp.float32),
            jax.ShapeDtypeStruct((NW, GA), jnp.float32),
        ],
        scratch_types=[
            pltpu.VMEM((GA,), jnp.float32),
            pltpu.VMEM((GA,), jnp.float32),
            pltpu.VMEM((NBB,), jnp.int32),
            pltpu.VMEM((NBB,), jnp.float32),
            pltpu.VMEM((16,), jnp.float32),
        ],
    )
    def k(al_hbm, be_hbm, ids_hbm, b_hbm, e_hbm, s_hbm,
          al_v, s_v, idx_v, be_v, b_v):
        w = _wid()
        base0 = w * NB
        pltpu.sync_copy(b_hbm, b_v)
        pltpu.sync_copy(al_hbm, al_v)
        b16 = b_v[...]

        def zero(i, _):
            s_v[pl.ds(i * L, L)] = jnp.zeros((L,), jnp.float32)
            return 0

        lax.fori_loop(0, GA // L, zero, 0, unroll=8)

        def blk(b, _):
            base = pl.multiple_of(base0 + b * NBB, 8)
            pltpu.sync_copy(ids_hbm.at[pl.ds(base, NBB)], idx_v)
            pltpu.sync_copy(be_hbm.at[pl.ds(base, NBB)], be_v)

            def grp(g, _):
                d16 = idx_v[pl.ds(g * L, L)]
                lg = plsc.load_gather(al_v, [d16]) + be_v[pl.ds(g * L, L)] + b16
                lg = jnp.minimum(_lrelu(lg), CLAMP)
                ev = jnp.exp(lg)
                be_v[pl.ds(g * L, L)] = ev
                plsc.addupdate_scatter(s_v, [d16], ev)
                return 0

            lax.fori_loop(0, NBB // L, grp, 0, unroll=2)
            pltpu.sync_copy(be_v, e_hbm.at[pl.ds(base, NBB)])
            return 0

        lax.fori_loop(0, nblk, blk, 0)
        pltpu.sync_copy(s_v, s_hbm.at[w])

    return k(alpha, beta, ids, bias16)


def _sc_acompute_ro(e, s1d, ids):
    nblk = NB // NBB

    @functools.partial(
        _sc_kernel,
        out_type=jax.ShapeDtypeStruct((NPAD,), jnp.float32),
        scratch_types=[
            pltpu.VMEM((GA,), jnp.float32),
            pltpu.VMEM((NBB,), jnp.int32),
            pltpu.VMEM((NBB,), jnp.float32),
        ],
    )
    def k(e_hbm, s_hbm, ids_hbm, a_hbm, s_v, idx_v, e_v):
        w = _wid()
        base0 = w * NB
        pltpu.sync_copy(s_hbm, s_v)

        def blk(b, _):
            base = pl.multiple_of(base0 + b * NBB, 8)
            pltpu.sync_copy(ids_hbm.at[pl.ds(base, NBB)], idx_v)
            pltpu.sync_copy(e_hbm.at[pl.ds(base, NBB)], e_v)

            def grp(g, _):
                d16 = idx_v[pl.ds(g * L, L)]
                sv = plsc.load_gather(s_v, [d16])
                e_v[pl.ds(g * L, L)] = e_v[pl.ds(g * L, L)] / sv
                return 0

            lax.fori_loop(0, NBB // L, grp, 0, unroll=2)
            pltpu.sync_copy(e_v, a_hbm.at[pl.ds(base, NBB)])
            return 0

        lax.fori_loop(0, nblk, blk, 0)

    return k(e, s1d, ids)


def _tc_node0(nf, WT, b, w1):
    BLK = 1024

    def body(nf_ref, wt_ref, b_ref, w1_ref, hv_ref, u_ref):
        y = jnp.dot(nf_ref[...], wt_ref[...],
                    preferred_element_type=jnp.float32) + b_ref[...]
        hv = _lrelu(y)
        hv_ref[...] = hv
        u_ref[...] = jnp.sum(hv * w1_ref[...], axis=1, keepdims=True)

    return pl.pallas_call(
        body,
        grid=(NPAD // BLK,),
        in_specs=[
            pl.BlockSpec((BLK, NF), lambda i: (i, 0)),
            pl.BlockSpec((NF, GF), lambda i: (0, 0)),
            pl.BlockSpec((1, GF), lambda i: (0, 0)),
            pl.BlockSpec((1, GF), lambda i: (0, 0)),
        ],
        out_specs=[
            pl.BlockSpec((BLK, GF), lambda i: (i, 0)),
            pl.BlockSpec((BLK, 1), lambda i: (i, 0)),
        ],
        out_shape=[
            jax.ShapeDtypeStruct((NPAD, GF), jnp.float32),
            jax.ShapeDtypeStruct((NPAD, 1), jnp.float32),
        ],
    )(nf, WT, b, w1)


def _tc_he1(nfs, ef, WT, b, w2):
    BLK = 2048

    def body(nfs_ref, ef_ref, wt_ref, b_ref, w2_ref, he_ref, t_ref):
        x = jnp.concatenate([nfs_ref[...], ef_ref[...]], axis=1)
        he1 = _lrelu(jnp.dot(x, wt_ref[...],
                             preferred_element_type=jnp.float32) + b_ref[...])
        he_ref[...] = he1
        t_ref[...] = jnp.sum(he1 * w2_ref[...], axis=1, keepdims=True)

    return pl.pallas_call(
        body,
        grid=(EPAD // BLK,),
        in_specs=[
            pl.BlockSpec((BLK, NF), lambda i: (i, 0)),
            pl.BlockSpec((BLK, EF), lambda i: (i, 0)),
            pl.BlockSpec((NF + EF, GF), lambda i: (0, 0)),
            pl.BlockSpec((1, GF), lambda i: (0, 0)),
            pl.BlockSpec((1, GF), lambda i: (0, 0)),
        ],
        out_specs=[
            pl.BlockSpec((BLK, GF), lambda i: (i, 0)),
            pl.BlockSpec((BLK, 1), lambda i: (i, 0)),
        ],
        out_shape=[
            jax.ShapeDtypeStruct((EPAD, GF), jnp.float32),
            jax.ShapeDtypeStruct((EPAD, 1), jnp.float32),
        ],
    )(nfs, ef, WT, b, w2)


def _tc_sumcols(s_all, width):
    BLK = 6272 if width == NPAD else width
    grid = width // BLK

    def body(s_ref, o_ref):
        o_ref[...] = jnp.sum(s_ref[...], axis=0, keepdims=True)

    return pl.pallas_call(
        body,
        grid=(grid,),
        in_specs=[pl.BlockSpec((NW, BLK), lambda i: (0, i))],
        out_specs=pl.BlockSpec((1, BLK), lambda i: (0, i)),
        out_shape=jax.ShapeDtypeStruct((1, width), jnp.float32),
    )(s_all)


def _gru_math(x, h, WihT, bih, WhhT, bhh):
    gi = jnp.dot(x, WihT, preferred_element_type=jnp.float32) + bih
    gh = jnp.dot(h, WhhT, preferred_element_type=jnp.float32) + bhh
    i_r, i_z, i_n = gi[:, :GF], gi[:, GF:2 * GF], gi[:, 2 * GF:]
    h_r, h_z, h_n = gh[:, :GF], gh[:, GF:2 * GF], gh[:, 2 * GF:]
    r = jax.nn.sigmoid(i_r + h_r)
    z = jax.nn.sigmoid(i_z + h_z)
    n = jnp.tanh(i_n + r * h_n)
    return (1.0 - z) * n + z * h


def _elu(c):
    return jnp.where(c > 0, c, jnp.exp(jnp.minimum(c, 0.0)) - 1.0)


def _tc_gru(cpart, h, pnWT, pnb, WihT, bih, WhhT, bhh, w1, w2):
    BLK = 1024

    def body(c_ref, h_ref, pnwt_ref, pnb_ref, wih_ref, bih_ref,
             whh_ref, bhh_ref, w1_ref, w2_ref, hn_ref, u_ref, v_ref):
        c = jnp.dot(c_ref[0] + c_ref[1], pnwt_ref[...],
                    preferred_element_type=jnp.float32) + pnb_ref[...]
        hn = jnp.maximum(_gru_math(_elu(c), h_ref[...], wih_ref[...],
                                   bih_ref[...], whh_ref[...], bhh_ref[...]),
                         0.0)
        hn_ref[...] = hn
        u_ref[...] = jnp.sum(hn * w1_ref[...], axis=1, keepdims=True)
        v_ref[...] = jnp.sum(hn * w2_ref[...], axis=1, keepdims=True)

    return pl.pallas_call(
        body,
        grid=(NPAD // BLK,),
        in_specs=[
            pl.BlockSpec((NC, BLK, GF), lambda i: (0, i, 0)),
            pl.BlockSpec((BLK, GF), lambda i: (i, 0)),
            pl.BlockSpec((GF, GF), lambda i: (0, 0)),
            pl.BlockSpec((1, GF), lambda i: (0, 0)),
            pl.BlockSpec((GF, 3 * GF), lambda i: (0, 0)),
            pl.BlockSpec((1, 3 * GF), lambda i: (0, 0)),
            pl.BlockSpec((GF, 3 * GF), lambda i: (0, 0)),
            pl.BlockSpec((1, 3 * GF), lambda i: (0, 0)),
            pl.BlockSpec((1, GF), lambda i: (0, 0)),
            pl.BlockSpec((1, GF), lambda i: (0, 0)),
        ],
        out_specs=[
            pl.BlockSpec((BLK, GF), lambda i: (i, 0)),
            pl.BlockSpec((BLK, 1), lambda i: (i, 0)),
            pl.BlockSpec((BLK, 1), lambda i: (i, 0)),
        ],
        out_shape=[
            jax.ShapeDtypeStruct((NPAD, GF), jnp.float32),
            jax.ShapeDtypeStruct((NPAD, 1), jnp.float32),
            jax.ShapeDtypeStruct((NPAD, 1), jnp.float32),
        ],
    )(cpart, h, pnWT, pnb, WihT, bih, WhhT, bhh, w1, w2)


def _tc_r1(g0, g1, w1):
    def body(g0_ref, g1_ref, w1_ref, gf_ref, al_ref):
        gf = g0_ref[...] + g1_ref[...]
        gf_ref[...] = gf
        al_ref[...] = jnp.sum(jnp.maximum(gf, 0.0) * w1_ref[...],
                              axis=1, keepdims=True)

    return pl.pallas_call(
        body,
        out_shape=[
            jax.ShapeDtypeStruct((G, GF), jnp.float32),
            jax.ShapeDtypeStruct((G, 1), jnp.float32),
        ],
    )(g0, g1, w1)


def _tc_gru_g(gr0, gr1, gf, pnWT, pnb, WihT, bih, WhhT, bhh):
    def body(g0_ref, g1_ref, gf_ref, pnwt_ref, pnb_ref, wih_ref, bih_ref,
             whh_ref, bhh_ref, o_ref):
        c = jnp.dot(g0_ref[...] + g1_ref[...], pnwt_ref[...],
                    preferred_element_type=jnp.float32) + pnb_ref[...]
        o_ref[...] = jnp.maximum(
            _gru_math(_elu(c), gf_ref[...], wih_ref[...], bih_ref[...],
                      whh_ref[...], bhh_ref[...]), 0.0)

    return pl.pallas_call(
        body,
        out_shape=jax.ShapeDtypeStruct((G, GF), jnp.float32),
    )(gr0, gr1, gf, pnWT, pnb, WihT, bih, WhhT, bhh)


def _tc_head(fpr, fp1T, fp1b, fp2T, fp2b, g2, pr1T, pr1b, pr2row, pr2bb):
    def body(fp_ref, w1_ref, b1_ref, w2_ref, b2_ref, g2_ref, p1_ref, pb1_ref,
             p2_ref, pb2_ref, o_ref):
        fph = jnp.maximum(jnp.dot(fp_ref[...], w1_ref[...],
                                  preferred_element_type=jnp.float32)
                          + b1_ref[...], 0.0)
        fpf = jnp.dot(fph, w2_ref[...],
                      preferred_element_type=jnp.float32) + b2_ref[...]
        comb = jnp.concatenate([g2_ref[...], fpf], axis=1)
        hh = jnp.maximum(jnp.dot(comb, p1_ref[...],
                                 preferred_element_type=jnp.float32)
                         + pb1_ref[...], 0.0)
        yy = hh * p2_ref[...] + pb2_ref[...]
        o_ref[...] = jnp.sum(yy, axis=1, keepdims=True)

    return pl.pallas_call(
        body,
        out_shape=jax.ShapeDtypeStruct((G, 1), jnp.float32),
    )(fpr, fp1T, fp1b, fp2T, fp2b, g2, pr1T, pr1b, pr2row, pr2bb)


def kernel(node_feats, edge_feats, fingerprints, params, edge_index, node2graph):
    p = params
    f32 = jnp.float32
    i32 = jnp.int32

    nf_pad = jnp.pad(node_feats, ((0, NPAD - N), (0, 0)))
    src = jnp.concatenate([edge_index[0], jnp.zeros((EPAD - E,), i32)])
    dst = jnp.concatenate([edge_index[1],
                           jnp.full((EPAD - E,), DUMMY_DST, i32)])
    ef_pad = jnp.pad(edge_feats, ((0, EPAD - E), (0, 0)))
    n2g = jnp.concatenate([node2graph, jnp.full((NPAD - N,), DUMMY_G, i32)])

    row = lambda x: x.reshape(1, -1)
    bvec = lambda s: jnp.broadcast_to(s.reshape(()), (16,)).astype(f32)

    gc_pe2_w1 = row(p['gc_pe2_W'][0, :GF])
    gc_pe2_w2 = row(p['gc_pe2_W'][0, GF:])
    l0_w1 = row(p['l0_pe_W'][0, :GF])
    l0_w2 = row(p['l0_pe_W'][0, GF:])
    l1_w1 = row(p['l1_pe_W'][0, :GF])
    l1_w2 = row(p['l1_pe_W'][0, GF:])
    cl_w1 = row(p['ro0_cl_W'][0, :GF])
    cl_w2 = row(p['ro0_cl_W'][0, GF:])

    hv, u0 = _tc_node0(nf_pad, p['gc_pn_W'].T, row(p['gc_pn_b']), gc_pe2_w1)
    nfs = _sc_gather_rows(src, node_feats)
    he1, t = _tc_he1(nfs, ef_pad, p['gc_pe1_W'].T, row(p['gc_pe1_b']),
                     gc_pe2_w2)
    e_gc, sall_gc = _sc_attention(u0.reshape(NPAD), dst, bvec(p['gc_pe2_b']),
                                  t=t.reshape(EPAD))
    s_gc = _tc_sumcols(sall_gc, NPAD).reshape(NPAD)
    a_gc = _sc_acompute(e_gc, s_gc, dst)
    c_gc = _sc_weighted_scatter(a_gc, dst, he1)
    h1, u1, v1 = _tc_gru(c_gc, hv, p['gc_et_W'].T, row(p['gc_et_b']),
                         p['gc_Wih'].T, row(p['gc_bih']), p['gc_Whh'].T,
                         row(p['gc_bhh']), l0_w1, l0_w2)

    hT1 = _sc_chunkify(h1)
    e0, sall0 = _sc_attention(u1.reshape(NPAD), dst, bvec(p['l0_pe_b']),
                              v=v1.reshape(NPAD), src=src)
    s0 = _tc_sumcols(sall0, NPAD).reshape(NPAD)
    a0 = _sc_acompute(e0, s0, dst)
    c0 = _sc_weighted_scatter(a0, dst, hT1, src=src)
    h2, u2, v2 = _tc_gru(c0, h1, p['l0_pn_W'].T, row(p['l0_pn_b']),
                         p['l0_Wih'].T, row(p['l0_bih']), p['l0_Whh'].T,
                         row(p['l0_bhh']), l1_w1, l1_w2)

    hT2 = _sc_chunkify(h2)
    e1, sall1 = _sc_attention(u2.reshape(NPAD), dst, bvec(p['l1_pe_b']),
                              v=v2.reshape(NPAD), src=src)
    s1 = _tc_sumcols(sall1, NPAD).reshape(NPAD)
    a1 = _sc_acompute(e1, s1, dst)
    c1 = _sc_weighted_scatter(a1, dst, hT2, src=src)
    h3, beta, _v3 = _tc_gru(c1, h2, p['l1_pn_W'].T, row(p['l1_pn_b']),
                            p['l1_Wih'].T, row(p['l1_bih']), p['l1_Whh'].T,
                            row(p['l1_bhh']), cl_w2, cl_w2)

    gt = _sc_segsum_rows(h3, n2g)
    gf, alpha = _tc_r1(gt[0, :G], gt[1, :G], cl_w1)
    alpha_pad = jnp.pad(alpha.reshape(G), (0, GA - G))
    e_ro, sall_ro = _sc_attention_ro(alpha_pad, beta.reshape(NPAD), n2g,
                                    bvec(p['ro0_cl_b']))
    s_ro = _tc_sumcols(sall_ro, GA).reshape(GA)
    an = _sc_acompute_ro(e_ro, s_ro, n2g)
    gr = _sc_segsum_rows(h3, n2g, weights=an)
    g2 = _tc_gru_g(gr[0, :G], gr[1, :G], gf, p['ro0_pn_W'].T,
                   row(p['ro0_pn_b']), p['ro0_Wih'].T, row(p['ro0_bih']),
                   p['ro0_Whh'].T, row(p['ro0_bhh']))

    pr2bb = jnp.broadcast_to(p['pr2_b'].reshape(()) / 128.0, (1, 128)).astype(f32)
    out = _tc_head(fingerprints, p['fp1_W'].T, row(p['fp1_b']), p['fp2_W'].T,
                   row(p['fp2_b']), g2, p['pr1_W'].T, row(p['pr1_b']),
                   row(p['pr2_W'][0]), pr2bb)
    return out
